# baseline (device time: 100183 ns/iter reference)
import jax
import jax.numpy as jnp
from jax import lax
from jax.experimental import pallas as pl
from jax.experimental.pallas import tpu as pltpu

N_DEV = 8
N_TOK = 512
D_IN = 256
D_OUT = 512
E_PER = 2


def kernel(x, router_W, route_idx, expert_W):
    del router_W

    def body(x_ref, idx_ref, w_ref, out_ref, comm_ref, send_sems, recv_sems):
        my = lax.axis_index("i")
        left = (my - 1) % N_DEV
        right = (my + 1) % N_DEV

        barrier_sem = pltpu.get_barrier_semaphore()
        for nbr in (left, right):
            pl.semaphore_signal(
                barrier_sem, inc=1,
                device_id=(nbr,), device_id_type=pl.DeviceIdType.MESH,
            )
        pl.semaphore_wait(barrier_sem, 2)

        e0 = (2 * my).astype(jnp.int32)
        idx = idx_ref[:, :]
        x_val = x_ref[:, :]
        x0 = jnp.where(idx == e0, x_val, 0.0)
        x1 = jnp.where(idx == e0 + 1, x_val, 0.0)
        x2 = jnp.concatenate([x0, x1], axis=1)
        wcat = w_ref[:, :, :].reshape(E_PER * D_IN, D_OUT)
        partial = jnp.dot(x2, wcat, preferred_element_type=jnp.float32)

        out_ref[:, :] = partial
        comm_ref[0, :, :] = partial

        for h in range(N_DEV - 1):
            rdma = pltpu.make_async_remote_copy(
                src_ref=comm_ref.at[h],
                dst_ref=comm_ref.at[h + 1],
                send_sem=send_sems.at[h],
                recv_sem=recv_sems.at[h],
                device_id=(right,),
                device_id_type=pl.DeviceIdType.MESH,
            )
            rdma.start()
            rdma.wait()
            out_ref[:, :] += comm_ref[h + 1, :, :]

    return pl.pallas_call(
        body,
        out_shape=jax.ShapeDtypeStruct((N_TOK, D_OUT), jnp.float32),
        in_specs=[
            pl.BlockSpec(memory_space=pltpu.VMEM),
            pl.BlockSpec(memory_space=pltpu.VMEM),
            pl.BlockSpec(memory_space=pltpu.VMEM),
        ],
        out_specs=pl.BlockSpec(memory_space=pltpu.VMEM),
        scratch_shapes=[
            pltpu.VMEM((N_DEV, N_TOK, D_OUT), jnp.float32),
            pltpu.SemaphoreType.DMA((N_DEV - 1,)),
            pltpu.SemaphoreType.DMA((N_DEV - 1,)),
        ],
        compiler_params=pltpu.CompilerParams(collective_id=0),
    )(x, route_idx, expert_W)


# device time: 39296 ns/iter; 2.5494x vs baseline; 2.5494x over previous
import jax
import jax.numpy as jnp
from jax import lax
from jax.experimental import pallas as pl
from jax.experimental.pallas import tpu as pltpu

N_DEV = 8
N_TOK = 512
D_IN = 256
D_OUT = 512
E_PER = 2

_RS = [(4, 256), (2, 128), (1, 64)]
_AG = [(1, 64), (2, 128), (4, 256)]


def kernel(x, router_W, route_idx, expert_W):
    del router_W

    def body(x_ref, idx_ref, w_ref, out_ref,
             recv0, recv1, recv2, rs_send, rs_recv, ag_send, ag_recv):
        my = lax.axis_index("i").astype(jnp.int32)

        barrier_sem = pltpu.get_barrier_semaphore()
        for k in (1, 2, 4):
            pl.semaphore_signal(
                barrier_sem, inc=1,
                device_id=(my ^ k,), device_id_type=pl.DeviceIdType.MESH,
            )
        pl.semaphore_wait(barrier_sem, 3)

        e0 = 2 * my
        idx = idx_ref[:, :]
        x_val = x_ref[:, :]
        x0 = jnp.where(idx == e0, x_val, 0.0)
        x1 = jnp.where(idx == e0 + 1, x_val, 0.0)
        x2 = jnp.concatenate([x0, x1], axis=1)
        wcat = w_ref[:, :, :].reshape(E_PER * D_IN, D_OUT)
        out_ref[:, :] = jnp.dot(x2, wcat, preferred_element_type=jnp.float32)

        recvs = [recv0, recv1, recv2]
        off = jnp.int32(0)
        for r, (k, sz) in enumerate(_RS):
            partner = my ^ k
            b = (my // k) % 2
            send_off = off + (1 - b) * sz
            keep_off = off + b * sz
            rdma = pltpu.make_async_remote_copy(
                src_ref=out_ref.at[pl.ds(send_off, sz), :],
                dst_ref=recvs[r],
                send_sem=rs_send.at[r],
                recv_sem=rs_recv.at[r],
                device_id=(partner,),
                device_id_type=pl.DeviceIdType.MESH,
            )
            rdma.start()
            rdma.wait()
            out_ref[pl.ds(keep_off, sz), :] = (
                out_ref[pl.ds(keep_off, sz), :] + recvs[r][:, :]
            )
            off = keep_off

        for r, (k, sz) in enumerate(_AG):
            partner = my ^ k
            b = (my // k) % 2
            rdma = pltpu.make_async_remote_copy(
                src_ref=out_ref.at[pl.ds(off, sz), :],
                dst_ref=out_ref.at[pl.ds(off, sz), :],
                send_sem=ag_send.at[r],
                recv_sem=ag_recv.at[r],
                device_id=(partner,),
                device_id_type=pl.DeviceIdType.MESH,
            )
            rdma.start()
            rdma.wait()
            off = off - b * sz

    return pl.pallas_call(
        body,
        out_shape=jax.ShapeDtypeStruct((N_TOK, D_OUT), jnp.float32),
        in_specs=[
            pl.BlockSpec(memory_space=pltpu.VMEM),
            pl.BlockSpec(memory_space=pltpu.VMEM),
            pl.BlockSpec(memory_space=pltpu.VMEM),
        ],
        out_specs=pl.BlockSpec(memory_space=pltpu.VMEM),
        scratch_shapes=[
            pltpu.VMEM((256, D_OUT), jnp.float32),
            pltpu.VMEM((128, D_OUT), jnp.float32),
            pltpu.VMEM((64, D_OUT), jnp.float32),
            pltpu.SemaphoreType.DMA((3,)),
            pltpu.SemaphoreType.DMA((3,)),
            pltpu.SemaphoreType.DMA((3,)),
            pltpu.SemaphoreType.DMA((3,)),
        ],
        compiler_params=pltpu.CompilerParams(collective_id=0),
    )(x, route_idx, expert_W)


# device time: 25564 ns/iter; 3.9189x vs baseline; 1.5372x over previous
import jax
import jax.numpy as jnp
from jax import lax
from jax.experimental import pallas as pl
from jax.experimental.pallas import tpu as pltpu

N_DEV = 8
N_TOK = 512
D_IN = 256
D_OUT = 512
E_PER = 2
C = N_TOK // N_DEV


def kernel(x, router_W, route_idx, expert_W):
    del router_W

    def body(x_ref, idx_ref, w_ref, out_ref,
             rs_buf, rs_send, rs_recv, ag_send, ag_recv):
        my = lax.axis_index("i").astype(jnp.int32)

        barrier_sem = pltpu.get_barrier_semaphore()
        for j in range(1, N_DEV):
            pl.semaphore_signal(
                barrier_sem, inc=1,
                device_id=((my + j) % N_DEV,),
                device_id_type=pl.DeviceIdType.MESH,
            )
        pl.semaphore_wait(barrier_sem, N_DEV - 1)

        e0 = 2 * my
        idx = idx_ref[:, :]
        x_val = x_ref[:, :]
        x0 = jnp.where(idx == e0, x_val, 0.0)
        x1 = jnp.where(idx == e0 + 1, x_val, 0.0)
        x2 = jnp.concatenate([x0, x1], axis=1)
        wcat = w_ref[:, :, :].reshape(E_PER * D_IN, D_OUT)
        out_ref[:, :] = jnp.dot(x2, wcat, preferred_element_type=jnp.float32)

        my_off = C * my

        rs_list = []
        for j in range(N_DEV - 1):
            t = (my + j + 1) % N_DEV
            rdma = pltpu.make_async_remote_copy(
                src_ref=out_ref.at[pl.ds(C * t, C), :],
                dst_ref=rs_buf.at[N_DEV - 2 - j],
                send_sem=rs_send.at[j],
                recv_sem=rs_recv.at[N_DEV - 2 - j],
                device_id=(t,),
                device_id_type=pl.DeviceIdType.MESH,
            )
            rdma.start()
            rs_list.append(rdma)
        for r in rs_list:
            r.wait_recv()

        acc = out_ref[pl.ds(my_off, C), :]
        for j in range(N_DEV - 1):
            acc = acc + rs_buf[j, :, :]
        out_ref[pl.ds(my_off, C), :] = acc

        ag_list = []
        for j in range(N_DEV - 1):
            t = (my + j + 1) % N_DEV
            rdma = pltpu.make_async_remote_copy(
                src_ref=out_ref.at[pl.ds(my_off, C), :],
                dst_ref=out_ref.at[pl.ds(my_off, C), :],
                send_sem=ag_send.at[j],
                recv_sem=ag_recv.at[N_DEV - 2 - j],
                device_id=(t,),
                device_id_type=pl.DeviceIdType.MESH,
            )
            rdma.start()
            ag_list.append(rdma)
        for r in ag_list:
            r.wait_recv()
        for r in rs_list:
            r.wait_send()
        for r in ag_list:
            r.wait_send()

    return pl.pallas_call(
        body,
        out_shape=jax.ShapeDtypeStruct((N_TOK, D_OUT), jnp.float32),
        in_specs=[
            pl.BlockSpec(memory_space=pltpu.VMEM),
            pl.BlockSpec(memory_space=pltpu.VMEM),
            pl.BlockSpec(memory_space=pltpu.VMEM),
        ],
        out_specs=pl.BlockSpec(memory_space=pltpu.VMEM),
        scratch_shapes=[
            pltpu.VMEM((N_DEV - 1, C, D_OUT), jnp.float32),
            pltpu.SemaphoreType.DMA((N_DEV - 1,)),
            pltpu.SemaphoreType.DMA((N_DEV - 1,)),
            pltpu.SemaphoreType.DMA((N_DEV - 1,)),
            pltpu.SemaphoreType.DMA((N_DEV - 1,)),
        ],
        compiler_params=pltpu.CompilerParams(collective_id=0),
    )(x, route_idx, expert_W)


# device time: 19869 ns/iter; 5.0422x vs baseline; 1.2866x over previous
import jax
import jax.numpy as jnp
from jax import lax
from jax.experimental import pallas as pl
from jax.experimental.pallas import tpu as pltpu

N_DEV = 8
N_TOK = 512
D_IN = 256
D_OUT = 512
E_PER = 2
C = N_TOK // N_DEV


def kernel(x, router_W, route_idx, expert_W):
    del router_W

    def body(x_ref, idx_ref, w_ref, out_ref,
             pbuf, rs_buf, agbuf, ag_buf,
             rs_send, rs_recv, ag_send, ag_recv):
        my = lax.axis_index("i").astype(jnp.int32)

        barrier_sem = pltpu.get_barrier_semaphore()
        for j in range(1, N_DEV):
            pl.semaphore_signal(
                barrier_sem, inc=1,
                device_id=((my + j) % N_DEV,),
                device_id_type=pl.DeviceIdType.MESH,
            )
        pl.semaphore_wait(barrier_sem, N_DEV - 1)

        e0 = 2 * my
        idx = idx_ref[:, :]
        x_val = x_ref[:, :]
        x0 = jnp.where(idx == e0, x_val, 0.0)
        x1 = jnp.where(idx == e0 + 1, x_val, 0.0)
        x2 = jnp.concatenate([x0, x1], axis=1)
        wcat = w_ref[:, :, :].reshape(E_PER * D_IN, D_OUT)
        partial = jnp.dot(
            x2.astype(jnp.bfloat16), wcat.astype(jnp.bfloat16),
            preferred_element_type=jnp.float32,
        )
        out_ref[:, :] = partial
        pbuf[:, :] = partial.astype(jnp.bfloat16)

        my_off = C * my

        rs_list = []
        for j in range(N_DEV - 1):
            t = (my + j + 1) % N_DEV
            rdma = pltpu.make_async_remote_copy(
                src_ref=pbuf.at[pl.ds(C * t, C), :],
                dst_ref=rs_buf.at[N_DEV - 2 - j],
                send_sem=rs_send.at[j],
                recv_sem=rs_recv.at[N_DEV - 2 - j],
                device_id=(t,),
                device_id_type=pl.DeviceIdType.MESH,
            )
            rdma.start()
            rs_list.append(rdma)
        for r in rs_list:
            r.wait_recv()

        acc = out_ref[pl.ds(my_off, C), :]
        for j in range(N_DEV - 1):
            acc = acc + rs_buf[j, :, :].astype(jnp.float32)
        out_ref[pl.ds(my_off, C), :] = acc
        agbuf[:, :] = acc.astype(jnp.bfloat16)

        ag_list = []
        for j in range(N_DEV - 1):
            t = (my + j + 1) % N_DEV
            rdma = pltpu.make_async_remote_copy(
                src_ref=agbuf,
                dst_ref=ag_buf.at[N_DEV - 2 - j],
                send_sem=ag_send.at[j],
                recv_sem=ag_recv.at[N_DEV - 2 - j],
                device_id=(t,),
                device_id_type=pl.DeviceIdType.MESH,
            )
            rdma.start()
            ag_list.append(rdma)
        for j in range(N_DEV - 1):
            r = ag_list[N_DEV - 2 - j]
            r.wait_recv()
            m = (my + j + 1) % N_DEV
            out_ref[pl.ds(C * m, C), :] = ag_buf[j, :, :].astype(jnp.float32)
        for r in rs_list:
            r.wait_send()
        for r in ag_list:
            r.wait_send()

    return pl.pallas_call(
        body,
        out_shape=jax.ShapeDtypeStruct((N_TOK, D_OUT), jnp.float32),
        in_specs=[
            pl.BlockSpec(memory_space=pltpu.VMEM),
            pl.BlockSpec(memory_space=pltpu.VMEM),
            pl.BlockSpec(memory_space=pltpu.VMEM),
        ],
        out_specs=pl.BlockSpec(memory_space=pltpu.VMEM),
        scratch_shapes=[
            pltpu.VMEM((N_TOK, D_OUT), jnp.bfloat16),
            pltpu.VMEM((N_DEV - 1, C, D_OUT), jnp.bfloat16),
            pltpu.VMEM((C, D_OUT), jnp.bfloat16),
            pltpu.VMEM((N_DEV - 1, C, D_OUT), jnp.bfloat16),
            pltpu.SemaphoreType.DMA((N_DEV - 1,)),
            pltpu.SemaphoreType.DMA((N_DEV - 1,)),
            pltpu.SemaphoreType.DMA((N_DEV - 1,)),
            pltpu.SemaphoreType.DMA((N_DEV - 1,)),
        ],
        compiler_params=pltpu.CompilerParams(collective_id=0),
    )(x, route_idx, expert_W)


# device time: 19236 ns/iter; 5.2081x vs baseline; 1.0329x over previous
import jax
import jax.numpy as jnp
from jax import lax
from jax.experimental import pallas as pl
from jax.experimental.pallas import tpu as pltpu

N_DEV = 8
N_TOK = 512
D_IN = 256
D_OUT = 512
E_PER = 2
C = N_TOK // N_DEV


def kernel(x, router_W, route_idx, expert_W):
    del router_W

    def body(x_ref, idx_ref, w_ref, out_ref,
             x2buf, pbuf, rs_buf, agbuf, ag_buf,
             rs_send, rs_recv, ag_send, ag_recv):
        my = lax.axis_index("i").astype(jnp.int32)

        barrier_sem = pltpu.get_barrier_semaphore()
        for j in range(1, N_DEV):
            pl.semaphore_signal(
                barrier_sem, inc=1,
                device_id=((my + j) % N_DEV,),
                device_id_type=pl.DeviceIdType.MESH,
            )

        e0 = 2 * my
        idx = idx_ref[:, :]
        x_val = x_ref[:, :]
        x0 = jnp.where(idx == e0, x_val, 0.0)
        x1 = jnp.where(idx == e0 + 1, x_val, 0.0)
        x2buf[:, :] = jnp.concatenate([x0, x1], axis=1).astype(jnp.bfloat16)
        wcat = w_ref[:, :, :].reshape(E_PER * D_IN, D_OUT).astype(jnp.bfloat16)

        def chunk_rows(t):
            return x2buf[pl.ds(C * t, C), :]

        pbuf[0, :, :] = jnp.dot(
            chunk_rows((my + 1) % N_DEV), wcat,
            preferred_element_type=jnp.float32,
        ).astype(jnp.bfloat16)

        pl.semaphore_wait(barrier_sem, N_DEV - 1)

        rs_list = []
        for j in range(N_DEV - 1):
            t = (my + j + 1) % N_DEV
            rdma = pltpu.make_async_remote_copy(
                src_ref=pbuf.at[j],
                dst_ref=rs_buf.at[N_DEV - 2 - j],
                send_sem=rs_send.at[j],
                recv_sem=rs_recv.at[N_DEV - 2 - j],
                device_id=(t,),
                device_id_type=pl.DeviceIdType.MESH,
            )
            rdma.start()
            rs_list.append(rdma)
            if j + 1 < N_DEV - 1:
                tn = (my + j + 2) % N_DEV
                pbuf[j + 1, :, :] = jnp.dot(
                    chunk_rows(tn), wcat, preferred_element_type=jnp.float32,
                ).astype(jnp.bfloat16)

        acc = jnp.dot(chunk_rows(my), wcat, preferred_element_type=jnp.float32)

        for j, r in enumerate(rs_list):
            r.wait_recv()
            acc = acc + rs_buf[N_DEV - 2 - j, :, :].astype(jnp.float32)
        my_off = C * my
        out_ref[pl.ds(my_off, C), :] = acc
        agbuf[:, :] = acc.astype(jnp.bfloat16)

        ag_list = []
        for j in range(N_DEV - 1):
            t = (my + j + 1) % N_DEV
            rdma = pltpu.make_async_remote_copy(
                src_ref=agbuf,
                dst_ref=ag_buf.at[N_DEV - 2 - j],
                send_sem=ag_send.at[j],
                recv_sem=ag_recv.at[N_DEV - 2 - j],
                device_id=(t,),
                device_id_type=pl.DeviceIdType.MESH,
            )
            rdma.start()
            ag_list.append(rdma)
        for j in range(N_DEV - 2, -1, -1):
            r = ag_list[N_DEV - 2 - j]
            r.wait_recv()
            m = (my + j + 1) % N_DEV
            out_ref[pl.ds(C * m, C), :] = ag_buf[j, :, :].astype(jnp.float32)
        for r in rs_list:
            r.wait_send()
        for r in ag_list:
            r.wait_send()

    return pl.pallas_call(
        body,
        out_shape=jax.ShapeDtypeStruct((N_TOK, D_OUT), jnp.float32),
        in_specs=[
            pl.BlockSpec(memory_space=pltpu.VMEM),
            pl.BlockSpec(memory_space=pltpu.VMEM),
            pl.BlockSpec(memory_space=pltpu.VMEM),
        ],
        out_specs=pl.BlockSpec(memory_space=pltpu.VMEM),
        scratch_shapes=[
            pltpu.VMEM((N_TOK, 2 * D_IN), jnp.bfloat16),
            pltpu.VMEM((N_DEV - 1, C, D_OUT), jnp.bfloat16),
            pltpu.VMEM((N_DEV - 1, C, D_OUT), jnp.bfloat16),
            pltpu.VMEM((C, D_OUT), jnp.bfloat16),
            pltpu.VMEM((N_DEV - 1, C, D_OUT), jnp.bfloat16),
            pltpu.SemaphoreType.DMA((N_DEV - 1,)),
            pltpu.SemaphoreType.DMA((N_DEV - 1,)),
            pltpu.SemaphoreType.DMA((N_DEV - 1,)),
            pltpu.SemaphoreType.DMA((N_DEV - 1,)),
        ],
        compiler_params=pltpu.CompilerParams(collective_id=0),
    )(x, route_idx, expert_W)
